# Initial kernel scaffold; baseline (speedup 1.0000x reference)
import jax
import jax.numpy as jnp
from jax.experimental import pallas as pl


def kernel(x, edge_index, adj_values, W):
    raise NotImplementedError("write your pallas kernel here")


# speedup vs baseline: 3.1911x; 3.1911x over previous
import functools

import jax
import jax.numpy as jnp
from jax import lax
from jax.experimental import pallas as pl
from jax.experimental.pallas import tpu as pltpu
from jax.experimental.pallas import tpu_sc as plsc

N = 10000
E = 320000
D = 128

NC = 2
NS = 16
L = 16

CH = 128
NCHUNK = 80
E_PAD = NC * NS * NCHUNK * CH
EROWS = E_PAD // CH
N_PAD = 10240
ROWS_PT = N_PAD // NS


def _mm_body(x_ref, w_ref, o_ref):
    o_ref[...] = lax.dot_general(
        x_ref[...], w_ref[...], (((1,), (1,)), ((), ())),
        preferred_element_type=jnp.float32)


def _matmul(x, W):
    grid = 10
    blk = N // grid
    return pl.pallas_call(
        _mm_body,
        grid=(grid,),
        in_specs=[
            pl.BlockSpec((blk, D), lambda i: (i, 0)),
            pl.BlockSpec((D, D), lambda i: (0, 0)),
        ],
        out_specs=pl.BlockSpec((blk, D), lambda i: (i, 0)),
        out_shape=jax.ShapeDtypeStruct((N, D), jnp.float32),
    )(x, W)


def _sc_body(xw_hbm, src_hbm, dst_hbm, adj_hbm, zero_hbm, out_hbm,
             src_v, dst_v, adj_v, rows_v, accum, sem):
    c = lax.axis_index("c")
    s = lax.axis_index("s")
    wid = s * NC + c
    base = wid * NCHUNK

    pltpu.sync_copy(zero_hbm.at[pl.ds(s * ROWS_PT, ROWS_PT)],
                    accum.at[pl.ds(s * ROWS_PT, ROWS_PT)])

    pltpu.sync_copy(src_hbm.at[pl.ds(base, NCHUNK)], src_v)
    pltpu.sync_copy(dst_hbm.at[pl.ds(base, NCHUNK)], dst_v)
    pltpu.sync_copy(adj_hbm.at[pl.ds(base, NCHUNK)], adj_v)

    plsc.subcore_barrier()

    def chunk(j, carry):
        pltpu.async_copy(xw_hbm.at[src_v.at[j]], rows_v, sem).wait()

        def grp(g16, cc):
            a16 = adj_v[j, pl.ds(g16 * L, L)]
            for i in range(L):
                e = g16 * L + i
                splat = a16.at[jnp.full((L,), i, jnp.int32)].get(
                    mode="promise_in_bounds")
                for g in range(D // L):
                    sl = pl.ds(g * L, L)
                    rows_v[e, sl] = rows_v[e, sl] * splat
            return cc
        lax.fori_loop(0, CH // L, grp, 0, unroll=False)

        pltpu.sync_copy(rows_v, accum.at[dst_v.at[j]], add=True)
        return carry

    lax.fori_loop(0, NCHUNK, chunk, 0, unroll=False)

    plsc.subcore_barrier()
    pltpu.sync_copy(accum.at[pl.ds(s * ROWS_PT, ROWS_PT)],
                    out_hbm.at[c, pl.ds(s * ROWS_PT, ROWS_PT)])


def _propagate(xw, src2d, dst2d, adj2d, zeros_init):
    mesh = plsc.VectorSubcoreMesh(core_axis_name="c", subcore_axis_name="s")
    k = pl.kernel(
        _sc_body,
        out_type=jax.ShapeDtypeStruct((NC, N_PAD, D), jnp.float32),
        mesh=mesh,
        scratch_types=[
            pltpu.VMEM((NCHUNK, CH), jnp.int32),
            pltpu.VMEM((NCHUNK, CH), jnp.int32),
            pltpu.VMEM((NCHUNK, CH), jnp.float32),
            pltpu.VMEM((CH, D), jnp.float32),
            pltpu.VMEM_SHARED((N_PAD, D), jnp.float32),
            pltpu.SemaphoreType.DMA,
        ],
    )
    return k(xw, src2d, dst2d, adj2d, zeros_init)


def _add_body(a_ref, b_ref, o_ref):
    o_ref[...] = a_ref[0] + b_ref[0]


def _combine(partial):
    grid = 25
    blk = N // grid
    return pl.pallas_call(
        _add_body,
        grid=(grid,),
        in_specs=[
            pl.BlockSpec((1, blk, D), lambda i: (0, i, 0)),
            pl.BlockSpec((1, blk, D), lambda i: (1, i, 0)),
        ],
        out_specs=pl.BlockSpec((blk, D), lambda i: (i, 0)),
        out_shape=jax.ShapeDtypeStruct((N, D), jnp.float32),
    )(partial, partial)


@jax.jit
def kernel(x, edge_index, adj_values, W):
    xw = _matmul(x, W)

    pad = E_PAD - E
    src = jnp.concatenate([edge_index[1], jnp.zeros((pad,), jnp.int32)])
    dst = jnp.concatenate([edge_index[0], jnp.full((pad,), N, jnp.int32)])
    adj = jnp.concatenate([adj_values, jnp.zeros((pad,), jnp.float32)])
    src2d = src.reshape(EROWS, CH)
    dst2d = dst.reshape(EROWS, CH)
    adj2d = adj.reshape(EROWS, CH)
    zeros_init = jnp.zeros((N_PAD, D), jnp.float32)

    partial = _propagate(xw, src2d, dst2d, adj2d, zeros_init)
    return _combine(partial)
